# Initial kernel scaffold; baseline (speedup 1.0000x reference)
import jax
import jax.numpy as jnp
from jax.experimental import pallas as pl


def kernel(trj_token, min_list, weekday_list, day_list, grid_list, poi_list, edge_index, feature, W1, a_src1, a_dst1, W2, a_src2, a_dst2, W3, grid_table, daytime_table, weekday_table, day_table):
    raise NotImplementedError("write your pallas kernel here")


# speedup vs baseline: 12.9493x; 12.9493x over previous
import functools
import math

import jax
import jax.numpy as jnp
import numpy as np
from jax import lax
from jax.experimental import pallas as pl
from jax.experimental.pallas import tpu as pltpu
from jax.experimental.pallas import tpu_sc as plsc

N = 10000
E = 160000
ET = 170000
D_MODEL = 768
H = 8
F1 = 16
F2 = 96
B = 64
L = 512
BL = B * L

NC = 2
NS = 16
NW = NC * NS
EPW = 5376
ETP = EPW * NW
ESTEPS = EPW // 128
ROWS_PT = N // NS

_i32 = jnp.int32
_f32 = jnp.float32


def _make_pe():
    pos = np.arange(L, dtype=np.float32)[:, None]
    div = np.exp(np.arange(0, D_MODEL, 2, dtype=np.float32)
                 * -(math.log(10000.0) / D_MODEL))
    pe = np.zeros((L, D_MODEL), dtype=np.float32)
    pe[:, 0::2] = np.sin(pos * div)
    pe[:, 1::2] = np.cos(pos * div)
    return pe


_PE = _make_pe()
_EXP1 = np.concatenate([np.kron(np.eye(H, dtype=np.float32),
                                np.ones((1, F1), np.float32)),
                        np.zeros((H, H * F1), np.float32)], axis=0)
_EXP2 = np.concatenate([np.kron(np.eye(H, dtype=np.float32),
                                np.ones((1, F2), np.float32)),
                        np.zeros((H, H * F2), np.float32)], axis=0)

_SC_MESH = plsc.VectorSubcoreMesh(core_axis_name="c", subcore_axis_name="s",
                                  num_cores=NC, num_subcores=NS)

_GDN = lax.GatherDimensionNumbers(offset_dims=(), collapsed_slice_dims=(0,),
                                  start_index_map=(0,))


def _vgather(v, idx):
    return lax.gather(v, idx[:, None], _GDN, (1,),
                      indices_are_sorted=False, unique_indices=False,
                      mode=lax.GatherScatterMode.PROMISE_IN_BOUNDS)


_RB = 2000


def _tc_layer_in(feature, w1, acomb1):
    def body(f_ref, w_ref, a_ref, hp_ref, esd_ref):
        hp = jnp.dot(f_ref[...], w_ref[...], preferred_element_type=_f32)
        hp_ref[...] = hp
        esd_ref[...] = jnp.dot(hp, a_ref[...], preferred_element_type=_f32)

    return pl.pallas_call(
        body,
        grid=(N // _RB,),
        in_specs=[
            pl.BlockSpec((_RB, 4), lambda i: (i, 0)),
            pl.BlockSpec((4, H * F1), lambda i: (0, 0)),
            pl.BlockSpec((H * F1, 16), lambda i: (0, 0)),
        ],
        out_specs=[
            pl.BlockSpec((_RB, H * F1), lambda i: (i, 0)),
            pl.BlockSpec((_RB, 16), lambda i: (i, 0)),
        ],
        out_shape=[
            jax.ShapeDtypeStruct((N, H * F1), _f32),
            jax.ShapeDtypeStruct((N, 16), _f32),
        ],
    )(feature, w1, acomb1)


def _tc_layer_mid(nump, denp, w2, acomb2, exp1):
    def body(n_ref, d_ref, w_ref, a_ref, e_ref, hp_ref, esd_ref):
        num = n_ref[0] + n_ref[1]
        den = d_ref[0] + d_ref[1]
        denb = jnp.dot(den, e_ref[...], preferred_element_type=_f32) + 1e-16
        h1 = num / denb
        h1 = jnp.where(h1 > 0, h1, (jnp.exp(h1) - 1.0))
        hp2 = jnp.dot(h1, w_ref[...], preferred_element_type=_f32)
        hp_ref[...] = hp2
        esd_ref[...] = jnp.dot(hp2, a_ref[...], preferred_element_type=_f32)

    return pl.pallas_call(
        body,
        grid=(N // _RB,),
        in_specs=[
            pl.BlockSpec((2, _RB, H * F1), lambda i: (0, i, 0)),
            pl.BlockSpec((2, _RB, 16), lambda i: (0, i, 0)),
            pl.BlockSpec((H * F1, H * F2), lambda i: (0, 0)),
            pl.BlockSpec((H * F2, 16), lambda i: (0, 0)),
            pl.BlockSpec((16, H * F1), lambda i: (0, 0)),
        ],
        out_specs=[
            pl.BlockSpec((_RB, H * F2), lambda i: (i, 0)),
            pl.BlockSpec((_RB, 16), lambda i: (i, 0)),
        ],
        out_shape=[
            jax.ShapeDtypeStruct((N, H * F2), _f32),
            jax.ShapeDtypeStruct((N, 16), _f32),
        ],
    )(nump, denp, w2, acomb2, exp1)


def _tc_layer_out(nump, denp, w3, exp2):
    def body(n_ref, d_ref, w_ref, e_ref, o_ref):
        num = n_ref[0] + n_ref[1]
        den = d_ref[0] + d_ref[1]
        denb = jnp.dot(den, e_ref[...], preferred_element_type=_f32) + 1e-16
        h2 = num / denb
        h2 = jnp.where(h2 > 0, h2, (jnp.exp(h2) - 1.0))
        o_ref[...] = jnp.dot(h2, w_ref[...], preferred_element_type=_f32)

    return pl.pallas_call(
        body,
        grid=(N // _RB,),
        in_specs=[
            pl.BlockSpec((2, _RB, H * F2), lambda i: (0, i, 0)),
            pl.BlockSpec((2, _RB, 16), lambda i: (0, i, 0)),
            pl.BlockSpec((H * F2, D_MODEL), lambda i: (0, 0)),
            pl.BlockSpec((16, H * F2), lambda i: (0, 0)),
        ],
        out_specs=pl.BlockSpec((_RB, D_MODEL), lambda i: (i, 0)),
        out_shape=jax.ShapeDtypeStruct((N, D_MODEL), _f32),
    )(nump, denp, w3, exp2)


def _rows_copy(src, dst, sid, soff, doff):
    base = sid * 624
    pltpu.sync_copy(src.at[pl.ds(soff + base, 624)],
                    dst.at[pl.ds(doff + base, 624)])

    @pl.when(sid == NS - 1)
    def _():
        rem = N - NS * 624
        pltpu.sync_copy(src.at[pl.ds(soff + NS * 624, rem)],
                        dst.at[pl.ds(doff + NS * 624, rem)])


def _sc_attention(esd, srcp, dstp, z16):
    def body(esd_h, srcp_h, dstp_h, z16_h, ee_h, denp_h,
             sidx, didx, sbuf, dbuf, eebuf, den_sh):
        cid = lax.axis_index("c")
        sid = lax.axis_index("s")
        wid = sid * NC + cid

        _rows_copy(z16_h, den_sh, sid, 0, 0)
        plsc.subcore_barrier()

        pltpu.sync_copy(srcp_h.at[wid], sidx)
        pltpu.sync_copy(dstp_h.at[wid], didx)

        def step(it, _):
            pltpu.sync_copy(esd_h.at[sidx.at[it]], sbuf)
            pltpu.sync_copy(esd_h.at[didx.at[it]], dbuf)
            base = (wid * ESTEPS + it) * 128

            def vec(k, _):
                lane = lax.iota(_i32, 16)
                ed_col = jnp.where(lane < 8, lane + 8, 15)
                es_v = sbuf[k, :]
                ed_v = _vgather(dbuf[k, :], ed_col)
                e = es_v + ed_v
                e = jnp.where(e >= 0, e, 0.2 * e)
                ee = jnp.exp(e)
                mf = jnp.where(lane < 8, 1.0, 0.0)
                mf = mf * jnp.where(jnp.full((16,), base + k, _i32) < ET,
                                    1.0, 0.0)
                eebuf[k, :] = ee * mf
                return 0

            lax.fori_loop(0, 128, vec, 0)
            pltpu.sync_copy(eebuf, ee_h.at[wid * ESTEPS + it])
            pltpu.sync_copy(eebuf, den_sh.at[didx.at[it]], add=True)
            return 0

        lax.fori_loop(0, ESTEPS, step, 0)
        plsc.subcore_barrier()

        _rows_copy(den_sh, denp_h, sid, 0, cid * N)

    run = pl.kernel(
        body,
        out_type=(
            jax.ShapeDtypeStruct((ETP // 128, 128, 16), _f32),
            jax.ShapeDtypeStruct((2 * N, 16), _f32),
        ),
        mesh=_SC_MESH,
        compiler_params=pltpu.CompilerParams(use_tc_tiling_on_sc=False),
        scratch_types=(
            pltpu.VMEM((ESTEPS, 128), _i32),
            pltpu.VMEM((ESTEPS, 128), _i32),
            pltpu.VMEM((128, 16), _f32),
            pltpu.VMEM((128, 16), _f32),
            pltpu.VMEM((128, 16), _f32),
            pltpu.VMEM_SHARED((N, 16), _f32),
        ),
    )
    return run(esd, srcp, dstp, z16)


def _sc_message(hp, ee, srcp, dstp, zf, feat_dim, col_base, fdim):
    nvec = feat_dim // 16
    vreg_head = [(col_base + k * 16) // fdim for k in range(nvec)]
    heads = sorted(set(vreg_head))

    def body(hp_h, ee_h, srcp_h, dstp_h, zf_h, nump_h,
             sidx, didx, eebuf, hpbuf, num_sh):
        cid = lax.axis_index("c")
        sid = lax.axis_index("s")
        wid = sid * NC + cid

        _rows_copy(zf_h, num_sh, sid, 0, 0)
        plsc.subcore_barrier()

        pltpu.sync_copy(srcp_h.at[wid], sidx)
        pltpu.sync_copy(dstp_h.at[wid], didx)

        def step(it, _):
            pltpu.sync_copy(ee_h.at[wid * ESTEPS + it], eebuf)
            pltpu.sync_copy(hp_h.at[sidx.at[it]], hpbuf)

            def edge(c, _):
                eerow = eebuf[c, :]
                spl = {}
                for hh in heads:
                    col = jnp.full((16,), hh, _i32)
                    spl[hh] = _vgather(eerow, col)
                for k in range(nvec):
                    sl = pl.ds(k * 16, 16)
                    hpbuf[c, sl] = hpbuf[c, sl] * spl[vreg_head[k]]
                return 0

            lax.fori_loop(0, 128, edge, 0)
            pltpu.sync_copy(hpbuf, num_sh.at[didx.at[it]], add=True)
            return 0

        lax.fori_loop(0, ESTEPS, step, 0)
        plsc.subcore_barrier()
        _rows_copy(num_sh, nump_h, sid, 0, cid * N)

    run = pl.kernel(
        body,
        out_type=jax.ShapeDtypeStruct((2 * N, feat_dim), _f32),
        mesh=_SC_MESH,
        compiler_params=pltpu.CompilerParams(use_tc_tiling_on_sc=False),
        scratch_types=(
            pltpu.VMEM((ESTEPS, 128), _i32),
            pltpu.VMEM((ESTEPS, 128), _i32),
            pltpu.VMEM((128, 16), _f32),
            pltpu.VMEM((128, feat_dim), _f32),
            pltpu.VMEM_SHARED((N, feat_dim), _f32),
        ),
    )
    return run(hp, ee, srcp, dstp, zf)


_TPS = BL // NW
_TCH = 16
_NST = _TPS // _TCH


def _sc_embed(node_emb, grid_t, dt_t, wk_t, day_t, pe, tok, gi, mi, wi, di):
    def body(node_h, grid_h, dt_h, wk_h, day_h, pe_h,
             tok_h, gi_h, mi_h, wi_h, di_h, out_h,
             itok, igr, imi, iwk, idy, b0, b1, b2, b3, b4, b5):
        cid = lax.axis_index("c")
        sid = lax.axis_index("s")
        wid = sid * NC + cid

        pltpu.sync_copy(tok_h.at[pl.ds(wid * _NST, _NST)], itok)
        pltpu.sync_copy(gi_h.at[pl.ds(wid * _NST, _NST)], igr)
        pltpu.sync_copy(mi_h.at[pl.ds(wid * _NST, _NST)], imi)
        pltpu.sync_copy(wi_h.at[pl.ds(wid * _NST, _NST)], iwk)
        pltpu.sync_copy(di_h.at[pl.ds(wid * _NST, _NST)], idy)

        def step(s, _):
            g = wid * _NST + s
            pltpu.sync_copy(node_h.at[itok.at[s]], b0)
            pltpu.sync_copy(grid_h.at[igr.at[s]], b1)
            pltpu.sync_copy(dt_h.at[imi.at[s]], b2)
            pltpu.sync_copy(wk_h.at[iwk.at[s]], b3)
            pltpu.sync_copy(day_h.at[idy.at[s]], b4)
            l0 = lax.rem(g * _TCH, L)
            pltpu.sync_copy(pe_h.at[pl.ds(l0, _TCH)], b5)

            def tokadd(t, _):
                for k in range(D_MODEL // 16):
                    sl = pl.ds(k * 16, 16)
                    b0[t, sl] = (b0[t, sl] + b1[t, sl] + b2[t, sl]
                                 + b3[t, sl] + b4[t, sl] + b5[t, sl])
                return 0

            lax.fori_loop(0, _TCH, tokadd, 0)
            pltpu.sync_copy(b0, out_h.at[pl.ds(g * _TCH, _TCH)])
            return 0

        lax.fori_loop(0, _NST, step, 0)

    run = pl.kernel(
        body,
        out_type=jax.ShapeDtypeStruct((BL, D_MODEL), _f32),
        mesh=_SC_MESH,
        compiler_params=pltpu.CompilerParams(use_tc_tiling_on_sc=False),
        scratch_types=(
            pltpu.VMEM((_NST, _TCH), _i32),
            pltpu.VMEM((_NST, _TCH), _i32),
            pltpu.VMEM((_NST, _TCH), _i32),
            pltpu.VMEM((_NST, _TCH), _i32),
            pltpu.VMEM((_NST, _TCH), _i32),
            pltpu.VMEM((_TCH, D_MODEL), _f32),
            pltpu.VMEM((_TCH, D_MODEL), _f32),
            pltpu.VMEM((_TCH, D_MODEL), _f32),
            pltpu.VMEM((_TCH, D_MODEL), _f32),
            pltpu.VMEM((_TCH, D_MODEL), _f32),
            pltpu.VMEM((_TCH, D_MODEL), _f32),
        ),
    )
    return run(node_emb, grid_t, dt_t, wk_t, day_t, pe, tok, gi, mi, wi, di)


def _attn_mat(a_src, a_dst, fdim):
    eye = jnp.eye(H, dtype=_f32)
    a_s = (a_src[:, :, None] * eye[:, None, :]).reshape(H * fdim, H)
    a_d = (a_dst[:, :, None] * eye[:, None, :]).reshape(H * fdim, H)
    return jnp.concatenate([a_s, a_d], axis=1)


def kernel(trj_token, min_list, weekday_list, day_list, grid_list, poi_list,
           edge_index, feature, W1, a_src1, a_dst1, W2, a_src2, a_dst2, W3,
           grid_table, daytime_table, weekday_table, day_table):
    del poi_list
    loop = jnp.arange(N, dtype=_i32)
    pad = jnp.zeros((ETP - ET,), _i32)
    srcp = jnp.concatenate([edge_index[0].astype(_i32), loop, pad]
                           ).reshape(NW, ESTEPS, 128)
    dstp = jnp.concatenate([edge_index[1].astype(_i32), loop, pad]
                           ).reshape(NW, ESTEPS, 128)

    acomb1 = _attn_mat(a_src1, a_dst1, F1)
    acomb2 = _attn_mat(a_src2, a_dst2, F2)
    exp1 = jnp.asarray(_EXP1)
    exp2 = jnp.asarray(_EXP2)
    pe = jnp.asarray(_PE)
    z16 = jnp.zeros((N, 16), _f32)
    z128 = jnp.zeros((N, 128), _f32)

    hp1, esd1 = _tc_layer_in(feature, W1, acomb1)
    ee1, den1p = _sc_attention(esd1, srcp, dstp, z16)
    num1p = _sc_message(hp1, ee1, srcp, dstp, z128, H * F1, 0, F1)

    hp2, esd2 = _tc_layer_mid(num1p.reshape(2, N, H * F1),
                              den1p.reshape(2, N, 16), W2, acomb2, exp1)
    ee2, den2p = _sc_attention(esd2, srcp, dstp, z16)
    num2_parts = []
    for j in range(6):
        hpj = lax.slice(hp2, (0, j * 128), (N, (j + 1) * 128))
        num2_parts.append(
            _sc_message(hpj, ee2, srcp, dstp, z128, 128, j * 128, F2)
            .reshape(2, N, 128))
    num2p = jnp.concatenate(num2_parts, axis=2)

    node_emb = _tc_layer_out(num2p, den2p.reshape(2, N, 16), W3, exp2)

    tok = trj_token.reshape(BL // _TCH, _TCH).astype(_i32)
    gi = grid_list.reshape(BL // _TCH, _TCH).astype(_i32)
    mi = min_list.reshape(BL // _TCH, _TCH).astype(_i32)
    wi = weekday_list.reshape(BL // _TCH, _TCH).astype(_i32)
    di = day_list.reshape(BL // _TCH, _TCH).astype(_i32)
    out = _sc_embed(node_emb, grid_table, daytime_table, weekday_table,
                    day_table, pe, tok, gi, mi, wi, di)
    return out.reshape(B, L, D_MODEL)


# speedup vs baseline: 18.6529x; 1.4405x over previous
import functools
import math

import jax
import jax.numpy as jnp
import numpy as np
from jax import lax
from jax.experimental import pallas as pl
from jax.experimental.pallas import tpu as pltpu
from jax.experimental.pallas import tpu_sc as plsc

N = 10000
E = 160000
ET = 170000
D_MODEL = 768
H = 8
F1 = 16
F2 = 96
B = 64
L = 512
BL = B * L

NC = 2
NS = 16
NW = NC * NS
EPW = 5376
ETP = EPW * NW
ESTEPS = EPW // 128
ROWS_PT = N // NS

_i32 = jnp.int32
_f32 = jnp.float32


def _make_pe():
    pos = np.arange(L, dtype=np.float32)[:, None]
    div = np.exp(np.arange(0, D_MODEL, 2, dtype=np.float32)
                 * -(math.log(10000.0) / D_MODEL))
    pe = np.zeros((L, D_MODEL), dtype=np.float32)
    pe[:, 0::2] = np.sin(pos * div)
    pe[:, 1::2] = np.cos(pos * div)
    return pe


_PE = _make_pe()
_EXP1 = np.concatenate([np.kron(np.eye(H, dtype=np.float32),
                                np.ones((1, F1), np.float32)),
                        np.zeros((H, H * F1), np.float32)], axis=0)
_EXP2 = np.concatenate([np.kron(np.eye(H, dtype=np.float32),
                                np.ones((1, F2), np.float32)),
                        np.zeros((H, H * F2), np.float32)], axis=0)

_SC_MESH = plsc.VectorSubcoreMesh(core_axis_name="c", subcore_axis_name="s",
                                  num_cores=NC, num_subcores=NS)

_GDN = lax.GatherDimensionNumbers(offset_dims=(), collapsed_slice_dims=(0,),
                                  start_index_map=(0,))


def _vgather(v, idx):
    return lax.gather(v, idx[:, None], _GDN, (1,),
                      indices_are_sorted=False, unique_indices=False,
                      mode=lax.GatherScatterMode.PROMISE_IN_BOUNDS)


_RB = 2000


def _tc_layer_in(feature, w1, acomb1):
    def body(f_ref, w_ref, a_ref, hp_ref, esd_ref):
        hp = jnp.dot(f_ref[...], w_ref[...], preferred_element_type=_f32)
        hp_ref[...] = hp
        esd_ref[...] = jnp.dot(hp, a_ref[...], preferred_element_type=_f32)

    return pl.pallas_call(
        body,
        grid=(N // _RB,),
        in_specs=[
            pl.BlockSpec((_RB, 4), lambda i: (i, 0)),
            pl.BlockSpec((4, H * F1), lambda i: (0, 0)),
            pl.BlockSpec((H * F1, 16), lambda i: (0, 0)),
        ],
        out_specs=[
            pl.BlockSpec((_RB, H * F1), lambda i: (i, 0)),
            pl.BlockSpec((_RB, 16), lambda i: (i, 0)),
        ],
        out_shape=[
            jax.ShapeDtypeStruct((N, H * F1), _f32),
            jax.ShapeDtypeStruct((N, 16), _f32),
        ],
    )(feature, w1, acomb1)


def _tc_layer_mid(nump, denp, w2, acomb2, exp1):
    def body(n_ref, d_ref, w_ref, a_ref, e_ref, hp_ref, esd_ref):
        num = n_ref[0] + n_ref[1]
        den = d_ref[0] + d_ref[1]
        denb = jnp.dot(den, e_ref[...], preferred_element_type=_f32) + 1e-16
        h1 = num / denb
        h1 = jnp.where(h1 > 0, h1, (jnp.exp(h1) - 1.0))
        hp2 = jnp.dot(h1, w_ref[...], preferred_element_type=_f32)
        hp_ref[...] = hp2
        esd_ref[...] = jnp.dot(hp2, a_ref[...], preferred_element_type=_f32)

    return pl.pallas_call(
        body,
        grid=(N // _RB,),
        in_specs=[
            pl.BlockSpec((2, _RB, H * F1), lambda i: (0, i, 0)),
            pl.BlockSpec((2, _RB, 16), lambda i: (0, i, 0)),
            pl.BlockSpec((H * F1, H * F2), lambda i: (0, 0)),
            pl.BlockSpec((H * F2, 16), lambda i: (0, 0)),
            pl.BlockSpec((16, H * F1), lambda i: (0, 0)),
        ],
        out_specs=[
            pl.BlockSpec((_RB, H * F2), lambda i: (i, 0)),
            pl.BlockSpec((_RB, 16), lambda i: (i, 0)),
        ],
        out_shape=[
            jax.ShapeDtypeStruct((N, H * F2), _f32),
            jax.ShapeDtypeStruct((N, 16), _f32),
        ],
    )(nump, denp, w2, acomb2, exp1)


def _tc_layer_out(nump, denp, w3, exp2):
    def body(n_ref, d_ref, w_ref, e_ref, o_ref):
        num = n_ref[0] + n_ref[1]
        den = d_ref[0] + d_ref[1]
        denb = jnp.dot(den, e_ref[...], preferred_element_type=_f32) + 1e-16
        h2 = num / denb
        h2 = jnp.where(h2 > 0, h2, (jnp.exp(h2) - 1.0))
        o_ref[...] = jnp.dot(h2, w_ref[...], preferred_element_type=_f32)

    return pl.pallas_call(
        body,
        grid=(N // _RB,),
        in_specs=[
            pl.BlockSpec((2, _RB, H * F2), lambda i: (0, i, 0)),
            pl.BlockSpec((2, _RB, 16), lambda i: (0, i, 0)),
            pl.BlockSpec((H * F2, D_MODEL), lambda i: (0, 0)),
            pl.BlockSpec((16, H * F2), lambda i: (0, 0)),
        ],
        out_specs=pl.BlockSpec((_RB, D_MODEL), lambda i: (i, 0)),
        out_shape=jax.ShapeDtypeStruct((N, D_MODEL), _f32),
    )(nump, denp, w3, exp2)


def _rows_copy(src, dst, sid, soff, doff):
    base = sid * 624
    pltpu.sync_copy(src.at[pl.ds(soff + base, 624)],
                    dst.at[pl.ds(doff + base, 624)])

    @pl.when(sid == NS - 1)
    def _():
        rem = N - NS * 624
        pltpu.sync_copy(src.at[pl.ds(soff + NS * 624, rem)],
                        dst.at[pl.ds(doff + NS * 624, rem)])


def _sc_attention(esd, srcp, dstp, z16):
    def body(esd_h, srcp_h, dstp_h, z16_h, ee_h, denp_h,
             sidx, didx, sbuf0, sbuf1, dbuf0, dbuf1, eebuf, den_sh,
             sem_s, sem_d):
        cid = lax.axis_index("c")
        sid = lax.axis_index("s")
        wid = sid * NC + cid

        _rows_copy(z16_h, den_sh, sid, 0, 0)
        plsc.subcore_barrier()

        pltpu.sync_copy(srcp_h.at[wid], sidx)
        pltpu.sync_copy(dstp_h.at[wid], didx)

        sbufs = (sbuf0, sbuf1)
        dbufs = (dbuf0, dbuf1)
        pltpu.async_copy(esd_h.at[sidx.at[0]], sbuf0, sem_s)
        pltpu.async_copy(esd_h.at[didx.at[0]], dbuf0, sem_d)

        def outer(io, _):
            for b2 in range(2):
                it = io * 2 + b2
                sbuf = sbufs[b2]
                dbuf = dbufs[b2]
                pltpu.make_async_copy(esd_h.at[sidx.at[it]], sbuf,
                                      sem_s).wait()
                pltpu.make_async_copy(esd_h.at[didx.at[it]], dbuf,
                                      sem_d).wait()
                itn = jnp.minimum(it + 1, ESTEPS - 1)
                pltpu.async_copy(esd_h.at[sidx.at[itn]], sbufs[1 - b2],
                                 sem_s)
                pltpu.async_copy(esd_h.at[didx.at[itn]], dbufs[1 - b2],
                                 sem_d)
                base = (wid * ESTEPS + it) * 128

                def vec(k, _):
                    lane = lax.iota(_i32, 16)
                    ed_col = jnp.where(lane < 8, lane + 8, 15)
                    es_v = sbuf[k, :]
                    ed_v = _vgather(dbuf[k, :], ed_col)
                    e = es_v + ed_v
                    e = jnp.where(e >= 0, e, 0.2 * e)
                    ee = jnp.exp(e)
                    mf = jnp.where(lane < 8, 1.0, 0.0)
                    mf = mf * jnp.where(jnp.full((16,), base + k, _i32) < ET,
                                        1.0, 0.0)
                    eebuf[k, :] = ee * mf
                    return 0

                lax.fori_loop(0, 128, vec, 0)
                pltpu.sync_copy(eebuf, ee_h.at[wid * ESTEPS + it])
                pltpu.sync_copy(eebuf, den_sh.at[didx.at[it]], add=True)
            return 0

        lax.fori_loop(0, ESTEPS // 2, outer, 0)
        pltpu.make_async_copy(esd_h.at[sidx.at[ESTEPS - 1]], sbuf0,
                              sem_s).wait()
        pltpu.make_async_copy(esd_h.at[didx.at[ESTEPS - 1]], dbuf0,
                              sem_d).wait()
        plsc.subcore_barrier()

        _rows_copy(den_sh, denp_h, sid, 0, cid * N)

    run = pl.kernel(
        body,
        out_type=(
            jax.ShapeDtypeStruct((ETP // 128, 128, 16), _f32),
            jax.ShapeDtypeStruct((2 * N, 16), _f32),
        ),
        mesh=_SC_MESH,
        compiler_params=pltpu.CompilerParams(use_tc_tiling_on_sc=False),
        scratch_types=(
            pltpu.VMEM((ESTEPS, 128), _i32),
            pltpu.VMEM((ESTEPS, 128), _i32),
            pltpu.VMEM((128, 16), _f32),
            pltpu.VMEM((128, 16), _f32),
            pltpu.VMEM((128, 16), _f32),
            pltpu.VMEM((128, 16), _f32),
            pltpu.VMEM((128, 16), _f32),
            pltpu.VMEM_SHARED((N, 16), _f32),
            pltpu.SemaphoreType.DMA,
            pltpu.SemaphoreType.DMA,
        ),
    )
    return run(esd, srcp, dstp, z16)


def _sc_message(hp, ee, srcp, dstp, zf, feat_dim, col_base, fdim):
    nvec = feat_dim // 16
    vreg_head = [(col_base + k * 16) // fdim for k in range(nvec)]
    heads = sorted(set(vreg_head))

    def body(hp_h, ee_h, srcp_h, dstp_h, zf_h, nump_h,
             sidx, didx, eebuf0, eebuf1, hpbuf0, hpbuf1, num_sh,
             sem_e, sem_h):
        cid = lax.axis_index("c")
        sid = lax.axis_index("s")
        wid = sid * NC + cid

        _rows_copy(zf_h, num_sh, sid, 0, 0)
        plsc.subcore_barrier()

        pltpu.sync_copy(srcp_h.at[wid], sidx)
        pltpu.sync_copy(dstp_h.at[wid], didx)

        eebufs = (eebuf0, eebuf1)
        hpbufs = (hpbuf0, hpbuf1)
        pltpu.async_copy(ee_h.at[wid * ESTEPS], eebuf0, sem_e)
        pltpu.async_copy(hp_h.at[sidx.at[0]], hpbuf0, sem_h)

        def outer(io, _):
            for b2 in range(2):
                it = io * 2 + b2
                eebuf = eebufs[b2]
                hpbuf = hpbufs[b2]
                pltpu.make_async_copy(ee_h.at[wid * ESTEPS + it], eebuf,
                                      sem_e).wait()
                pltpu.make_async_copy(hp_h.at[sidx.at[it]], hpbuf,
                                      sem_h).wait()
                itn = jnp.minimum(it + 1, ESTEPS - 1)
                pltpu.async_copy(ee_h.at[wid * ESTEPS + itn], eebufs[1 - b2],
                                 sem_e)
                pltpu.async_copy(hp_h.at[sidx.at[itn]], hpbufs[1 - b2],
                                 sem_h)

                def edge(c, _):
                    eerow = eebuf[c, :]
                    spl = {}
                    for hh in heads:
                        col = jnp.full((16,), hh, _i32)
                        spl[hh] = _vgather(eerow, col)
                    for k in range(nvec):
                        sl = pl.ds(k * 16, 16)
                        hpbuf[c, sl] = hpbuf[c, sl] * spl[vreg_head[k]]
                    return 0

                lax.fori_loop(0, 128, edge, 0)
                pltpu.sync_copy(hpbuf, num_sh.at[didx.at[it]], add=True)
            return 0

        lax.fori_loop(0, ESTEPS // 2, outer, 0)
        pltpu.make_async_copy(ee_h.at[wid * ESTEPS + ESTEPS - 1], eebuf0,
                              sem_e).wait()
        pltpu.make_async_copy(hp_h.at[sidx.at[ESTEPS - 1]], hpbuf0,
                              sem_h).wait()
        plsc.subcore_barrier()
        _rows_copy(num_sh, nump_h, sid, 0, cid * N)

    run = pl.kernel(
        body,
        out_type=jax.ShapeDtypeStruct((2 * N, feat_dim), _f32),
        mesh=_SC_MESH,
        compiler_params=pltpu.CompilerParams(use_tc_tiling_on_sc=False),
        scratch_types=(
            pltpu.VMEM((ESTEPS, 128), _i32),
            pltpu.VMEM((ESTEPS, 128), _i32),
            pltpu.VMEM((128, 16), _f32),
            pltpu.VMEM((128, 16), _f32),
            pltpu.VMEM((128, feat_dim), _f32),
            pltpu.VMEM((128, feat_dim), _f32),
            pltpu.VMEM_SHARED((N, feat_dim), _f32),
            pltpu.SemaphoreType.DMA,
            pltpu.SemaphoreType.DMA,
        ),
    )
    return run(hp, ee, srcp, dstp, zf)


_TPS = BL // NW
_TCH = 8
_NST = _TPS // _TCH


def _sc_embed(node_emb, grid_t, dt_t, wk_t, day_t, pe, tok, gi, mi, wi, di):
    def body(node_h, grid_h, dt_h, wk_h, day_h, pe_h,
             tok_h, gi_h, mi_h, wi_h, di_h, out_h,
             itok, igr, imi, iwk, idy,
             a0, a1, a2, a3, a4, a5, c0, c1, c2, c3, c4, c5, sem):
        cid = lax.axis_index("c")
        sid = lax.axis_index("s")
        wid = sid * NC + cid

        pltpu.sync_copy(tok_h.at[pl.ds(wid * _NST, _NST)], itok)
        pltpu.sync_copy(gi_h.at[pl.ds(wid * _NST, _NST)], igr)
        pltpu.sync_copy(mi_h.at[pl.ds(wid * _NST, _NST)], imi)
        pltpu.sync_copy(wi_h.at[pl.ds(wid * _NST, _NST)], iwk)
        pltpu.sync_copy(di_h.at[pl.ds(wid * _NST, _NST)], idy)

        bufsets = ((a0, a1, a2, a3, a4, a5), (c0, c1, c2, c3, c4, c5))

        def _descs(s, bufs):
            g = wid * _NST + s
            l0 = lax.rem(g * _TCH, L)
            return (
                (node_h.at[itok.at[s]], bufs[0]),
                (grid_h.at[igr.at[s]], bufs[1]),
                (dt_h.at[imi.at[s]], bufs[2]),
                (wk_h.at[iwk.at[s]], bufs[3]),
                (day_h.at[idy.at[s]], bufs[4]),
                (pe_h.at[pl.ds(l0, _TCH)], bufs[5]),
            )

        for src, dst in _descs(0, bufsets[0]):
            pltpu.async_copy(src, dst, sem)

        def outer(io, _):
            for b2 in range(2):
                s = io * 2 + b2
                bufs = bufsets[b2]
                for src, dst in _descs(s, bufs):
                    pltpu.make_async_copy(src, dst, sem).wait()
                sn = jnp.minimum(s + 1, _NST - 1)
                for src, dst in _descs(sn, bufsets[1 - b2]):
                    pltpu.async_copy(src, dst, sem)
                g = wid * _NST + s

                def tokadd(t, _):
                    for k in range(D_MODEL // 16):
                        sl = pl.ds(k * 16, 16)
                        bufs[0][t, sl] = (bufs[0][t, sl] + bufs[1][t, sl]
                                          + bufs[2][t, sl] + bufs[3][t, sl]
                                          + bufs[4][t, sl] + bufs[5][t, sl])
                    return 0

                lax.fori_loop(0, _TCH, tokadd, 0)
                pltpu.sync_copy(bufs[0], out_h.at[pl.ds(g * _TCH, _TCH)])
            return 0

        lax.fori_loop(0, _NST // 2, outer, 0)
        for src, dst in _descs(_NST - 1, bufsets[0]):
            pltpu.make_async_copy(src, dst, sem).wait()

    run = pl.kernel(
        body,
        out_type=jax.ShapeDtypeStruct((BL, D_MODEL), _f32),
        mesh=_SC_MESH,
        compiler_params=pltpu.CompilerParams(use_tc_tiling_on_sc=False),
        scratch_types=(
            pltpu.VMEM((_NST, _TCH), _i32),
            pltpu.VMEM((_NST, _TCH), _i32),
            pltpu.VMEM((_NST, _TCH), _i32),
            pltpu.VMEM((_NST, _TCH), _i32),
            pltpu.VMEM((_NST, _TCH), _i32),
        ) + tuple(pltpu.VMEM((_TCH, D_MODEL), _f32) for _ in range(12))
        + (pltpu.SemaphoreType.DMA,),
    )
    return run(node_emb, grid_t, dt_t, wk_t, day_t, pe, tok, gi, mi, wi, di)


def _attn_mat(a_src, a_dst, fdim):
    eye = jnp.eye(H, dtype=_f32)
    a_s = (a_src[:, :, None] * eye[:, None, :]).reshape(H * fdim, H)
    a_d = (a_dst[:, :, None] * eye[:, None, :]).reshape(H * fdim, H)
    return jnp.concatenate([a_s, a_d], axis=1)


def kernel(trj_token, min_list, weekday_list, day_list, grid_list, poi_list,
           edge_index, feature, W1, a_src1, a_dst1, W2, a_src2, a_dst2, W3,
           grid_table, daytime_table, weekday_table, day_table):
    del poi_list
    loop = jnp.arange(N, dtype=_i32)
    pad = jnp.zeros((ETP - ET,), _i32)
    srcp = jnp.concatenate([edge_index[0].astype(_i32), loop, pad]
                           ).reshape(NW, ESTEPS, 128)
    dstp = jnp.concatenate([edge_index[1].astype(_i32), loop, pad]
                           ).reshape(NW, ESTEPS, 128)

    acomb1 = _attn_mat(a_src1, a_dst1, F1)
    acomb2 = _attn_mat(a_src2, a_dst2, F2)
    exp1 = jnp.asarray(_EXP1)
    exp2 = jnp.asarray(_EXP2)
    pe = jnp.asarray(_PE)
    z16 = jnp.zeros((N, 16), _f32)
    z128 = jnp.zeros((N, 128), _f32)

    hp1, esd1 = _tc_layer_in(feature, W1, acomb1)
    ee1, den1p = _sc_attention(esd1, srcp, dstp, z16)
    num1p = _sc_message(hp1, ee1, srcp, dstp, z128, H * F1, 0, F1)

    hp2, esd2 = _tc_layer_mid(num1p.reshape(2, N, H * F1),
                              den1p.reshape(2, N, 16), W2, acomb2, exp1)
    ee2, den2p = _sc_attention(esd2, srcp, dstp, z16)
    num2_parts = []
    for j in range(6):
        hpj = lax.slice(hp2, (0, j * 128), (N, (j + 1) * 128))
        num2_parts.append(
            _sc_message(hpj, ee2, srcp, dstp, z128, 128, j * 128, F2)
            .reshape(2, N, 128))
    num2p = jnp.concatenate(num2_parts, axis=2)

    node_emb = _tc_layer_out(num2p, den2p.reshape(2, N, 16), W3, exp2)

    tok = trj_token.reshape(BL // _TCH, _TCH).astype(_i32)
    gi = grid_list.reshape(BL // _TCH, _TCH).astype(_i32)
    mi = min_list.reshape(BL // _TCH, _TCH).astype(_i32)
    wi = weekday_list.reshape(BL // _TCH, _TCH).astype(_i32)
    di = day_list.reshape(BL // _TCH, _TCH).astype(_i32)
    out = _sc_embed(node_emb, grid_table, daytime_table, weekday_table,
                    day_table, pe, tok, gi, mi, wi, di)
    return out.reshape(B, L, D_MODEL)
